# baseline (speedup 1.0000x reference)
import jax
import jax.numpy as jnp
from jax import lax
from jax.experimental import pallas as pl
from jax.experimental.pallas import tpu as pltpu
from jax.experimental.pallas import tpu_sc as plsc

H = 128
NH = 8
HD = H // NH
NT = 1024
SCALE = HD ** (-0.5)
NC = 100000
BC = 2000
NB = 50

NW = 32
SCH = 80
SNCH = NC // SCH


def _sc_scatter_body(exv_hbm, idx_hbm, zatt_hbm, att_out,
                     idx_v, rows_v, att_sh):
    c = lax.axis_index("c")
    s = lax.axis_index("s")
    w = s * 2 + c

    @pl.when(s == 0)
    def _():
        pltpu.sync_copy(zatt_hbm, att_sh)

    plsc.subcore_barrier()

    nk = 39 + jnp.where(w < SNCH % NW, 1, 0)

    def body(k, carry):
        off = (k * NW + w) * SCH
        pltpu.sync_copy(idx_hbm.at[pl.ds(off, SCH)], idx_v.at[0])
        pltpu.sync_copy(exv_hbm.at[pl.ds(off, SCH)], rows_v)
        pltpu.sync_copy(rows_v, att_sh.at[idx_v.at[0]], add=True)
        return carry

    lax.fori_loop(0, nk, body, 0)

    plsc.subcore_barrier()

    @pl.when(s == 0)
    def _():
        pltpu.sync_copy(att_sh, att_out.at[c])


def _sc_scatter(exv, idx):
    mesh = plsc.VectorSubcoreMesh(
        core_axis_name="c", subcore_axis_name="s",
        num_cores=2, num_subcores=16)
    zatt = jnp.zeros((NT, H), jnp.float32)
    return pl.kernel(
        _sc_scatter_body,
        out_type=jax.ShapeDtypeStruct((2, NT, H), jnp.float32),
        mesh=mesh,
        scratch_types=[
            pltpu.VMEM((1, SCH), jnp.int32),
            pltpu.VMEM((SCH, H), jnp.float32),
            pltpu.VMEM_SHARED((NT, H), jnp.float32),
        ],
    )(exv, idx, zatt)


def _ln(x, g, b):
    mu = jnp.mean(x, axis=-1, keepdims=True)
    var = jnp.mean((x - mu) ** 2, axis=-1, keepdims=True)
    return (x - mu) * jax.lax.rsqrt(var + 1e-5) * g + b


def _head_expand():
    r = jax.lax.broadcasted_iota(jnp.int32, (NH, H), 0)
    c = jax.lax.broadcasted_iota(jnp.int32, (NH, H), 1)
    return (r == c // HD).astype(jnp.float32)


def _exv_kernel(cell_ref, lab_ref, tis_ref, wqT_ref, bq_ref,
                wkT_ref, bk_ref, wvT_ref, bv_ref,
                exv_ref, dn_ref, dn_scr, q_scr):
    i = pl.program_id(0)

    @pl.when(i == 0)
    def _():
        dn_scr[...] = jnp.zeros_like(dn_scr)
        q_scr[...] = (
            jnp.dot(tis_ref[...], wqT_ref[...],
                    preferred_element_type=jnp.float32) + bq_ref[...]
        ).astype(jnp.bfloat16)

    cell_bf = cell_ref[...].astype(jnp.bfloat16)
    K = (jnp.dot(cell_bf, wkT_ref[...].astype(jnp.bfloat16),
                 preferred_element_type=jnp.float32) + bk_ref[...])
    lab = lab_ref[0]
    tid = jax.lax.broadcasted_iota(jnp.int32, (1, NT), 1)
    oh = (lab == tid).astype(jnp.bfloat16)
    qg = jnp.dot(oh, q_scr[...],
                 preferred_element_type=jnp.float32)
    s = jnp.dot(qg * K, _head_expand().T,
                preferred_element_type=jnp.float32) * SCALE
    ex = jnp.exp(s)
    V = (jnp.dot(cell_bf, wvT_ref[...].astype(jnp.bfloat16),
                 preferred_element_type=jnp.float32) + bv_ref[...])
    exR = jnp.dot(ex, _head_expand(),
                  preferred_element_type=jnp.float32)
    exv_ref[...] = V * exR
    sel = (jax.lax.broadcasted_iota(jnp.int32, (NH, 16), 0)
           == jax.lax.broadcasted_iota(jnp.int32, (NH, 16), 1)
           ).astype(jnp.float32)
    col8 = (jax.lax.broadcasted_iota(jnp.int32, (BC, 16), 1) == 8
            ).astype(jnp.float32)
    dnr = jnp.dot(ex, sel, preferred_element_type=jnp.float32) + col8
    dn_scr[...] += jax.lax.dot_general(
        oh, dnr.astype(jnp.bfloat16), (((0,), (0,)), ((), ())),
        preferred_element_type=jnp.float32)

    @pl.when(i == NB - 1)
    def _():
        dn_ref[...] = dn_scr[...]


def _cell_kernel(cell_ref, lab_ref, att2_ref, dn_ref, tis_ref,
                 woT_ref, bo_ref, tdwvT_ref, tdbv_ref, tdwoT_ref, tdbo_ref,
                 tg_ref, tb_ref, g_ref, b_ref,
                 tout_ref, out_ref, td_scr):
    i = pl.program_id(0)

    @pl.when(i == 0)
    def _():
        dn = dn_ref[...]
        att_raw = att2_ref[0] + att2_ref[1]
        counts = dn[:, 8:9]
        mask = counts > 0.5
        denom = dn[:, :NH]
        denom = jnp.where(denom == 0.0, 1.0, denom)
        rep = jnp.dot(1.0 / denom, _head_expand(),
                      preferred_element_type=jnp.float32)
        att = att_raw * rep
        att_o = (jnp.dot(att, woT_ref[...],
                         preferred_element_type=jnp.float32) + bo_ref[...])
        tis = tis_ref[...]
        t_upd = jnp.where(mask, att_o, tis)
        td_v = (jnp.dot(t_upd, tdwvT_ref[...],
                        preferred_element_type=jnp.float32) + tdbv_ref[...])
        td_scr[...] = (jnp.dot(td_v, tdwoT_ref[...],
                               preferred_element_type=jnp.float32)
                       + tdbo_ref[...]).astype(jnp.bfloat16)
        tout_ref[...] = _ln(tis + t_upd, tg_ref[...], tb_ref[...])

    lab = lab_ref[0]
    tid = jax.lax.broadcasted_iota(jnp.int32, (1, NT), 1)
    oh = (lab == tid).astype(jnp.bfloat16)
    G = jnp.dot(oh, td_scr[...], preferred_element_type=jnp.float32)
    out_ref[...] = _ln(cell_ref[...] + G, g_ref[...], b_ref[...])


def _full(shape):
    return pl.BlockSpec(shape, lambda i: tuple(0 for _ in shape))


def kernel(cell_features, tissue_features, cluster_labels, tissue_batch,
           bu_Wq, bu_bq, bu_Wk, bu_bk, bu_Wv, bu_bv, bu_Wo, bu_bo,
           td_Wq, td_bq, td_Wk, td_bk, td_Wv, td_bv, td_Wo, td_bo,
           cell_ln_g, cell_ln_b, tissue_ln_g, tissue_ln_b):
    lab_col = cluster_labels.reshape(NB, BC, 1)

    exv, dn = pl.pallas_call(
        _exv_kernel,
        grid=(NB,),
        in_specs=[
            pl.BlockSpec((BC, H), lambda i: (i, 0)),
            pl.BlockSpec((1, BC, 1), lambda i: (i, 0, 0)),
            _full((NT, H)),
            _full((H, H)),
            _full((1, H)),
            _full((H, H)),
            _full((1, H)),
            _full((H, H)),
            _full((1, H)),
        ],
        out_specs=[
            pl.BlockSpec((BC, H), lambda i: (i, 0)),
            pl.BlockSpec((NT, 16), lambda i: (0, 0)),
        ],
        out_shape=[
            jax.ShapeDtypeStruct((NC, H), jnp.float32),
            jax.ShapeDtypeStruct((NT, 16), jnp.float32),
        ],
        scratch_shapes=[
            pltpu.VMEM((NT, 16), jnp.float32),
            pltpu.VMEM((NT, H), jnp.bfloat16),
        ],
    )(cell_features, lab_col, tissue_features,
      bu_Wq.T, bu_bq.reshape(1, H),
      bu_Wk.T, bu_bk.reshape(1, H),
      bu_Wv.T, bu_bv.reshape(1, H))

    att2 = _sc_scatter(exv, cluster_labels)

    tissue_out, cell_out = pl.pallas_call(
        _cell_kernel,
        grid=(NB,),
        in_specs=[
            pl.BlockSpec((BC, H), lambda i: (i, 0)),
            pl.BlockSpec((1, BC, 1), lambda i: (i, 0, 0)),
            _full((2, NT, H)),
            _full((NT, 16)),
            _full((NT, H)),
            _full((H, H)),
            _full((1, H)),
            _full((H, H)),
            _full((1, H)),
            _full((H, H)),
            _full((1, H)),
            _full((1, H)),
            _full((1, H)),
            _full((1, H)),
            _full((1, H)),
        ],
        out_specs=[
            pl.BlockSpec((NT, H), lambda i: (0, 0)),
            pl.BlockSpec((BC, H), lambda i: (i, 0)),
        ],
        out_shape=[
            jax.ShapeDtypeStruct((NT, H), jnp.float32),
            jax.ShapeDtypeStruct((NC, H), jnp.float32),
        ],
        scratch_shapes=[pltpu.VMEM((NT, H), jnp.bfloat16)],
    )(cell_features, lab_col, att2, dn, tissue_features,
      bu_Wo.T, bu_bo.reshape(1, H),
      td_Wv.T, td_bv.reshape(1, H), td_Wo.T, td_bo.reshape(1, H),
      tissue_ln_g.reshape(1, H), tissue_ln_b.reshape(1, H),
      cell_ln_g.reshape(1, H), cell_ln_b.reshape(1, H))

    return cell_out, tissue_out


# speedup vs baseline: 1.0462x; 1.0462x over previous
import jax
import jax.numpy as jnp
from jax import lax
from jax.experimental import pallas as pl
from jax.experimental.pallas import tpu as pltpu
from jax.experimental.pallas import tpu_sc as plsc

H = 128
NH = 8
HD = H // NH
NT = 1024
SCALE = HD ** (-0.5)
NC = 100000
BC = 2000
NB = 50

NW = 32
SCH = 80
SNCH = NC // SCH


def _sc_scatter_body(exv_hbm, idx_hbm, zatt_hbm, att_out,
                     idx_v, rows_v, att_sh):
    c = lax.axis_index("c")
    s = lax.axis_index("s")
    w = s * 2 + c

    @pl.when(s == 0)
    def _():
        pltpu.sync_copy(zatt_hbm, att_sh)

    plsc.subcore_barrier()

    nk = 39 + jnp.where(w < SNCH % NW, 1, 0)

    def body(k, carry):
        off = (k * NW + w) * SCH
        pltpu.sync_copy(idx_hbm.at[pl.ds(off, SCH)], idx_v.at[0])
        pltpu.sync_copy(exv_hbm.at[pl.ds(off, SCH)], rows_v)
        pltpu.sync_copy(rows_v, att_sh.at[idx_v.at[0]], add=True)
        return carry

    lax.fori_loop(0, nk, body, 0)

    plsc.subcore_barrier()

    @pl.when(s == 0)
    def _():
        pltpu.sync_copy(att_sh, att_out.at[c])


def _sc_scatter(exv, idx):
    mesh = plsc.VectorSubcoreMesh(
        core_axis_name="c", subcore_axis_name="s",
        num_cores=2, num_subcores=16)
    zatt = jnp.zeros((NT, H), jnp.float32)
    return pl.kernel(
        _sc_scatter_body,
        out_type=jax.ShapeDtypeStruct((2, NT, H), jnp.float32),
        mesh=mesh,
        scratch_types=[
            pltpu.VMEM((1, SCH), jnp.int32),
            pltpu.VMEM((SCH, H), jnp.float32),
            pltpu.VMEM_SHARED((NT, H), jnp.float32),
        ],
    )(exv, idx, zatt)


def _ln(x, g, b):
    mu = jnp.mean(x, axis=-1, keepdims=True)
    var = jnp.mean((x - mu) ** 2, axis=-1, keepdims=True)
    return (x - mu) * jax.lax.rsqrt(var + 1e-5) * g + b


def _head_expand():
    r = jax.lax.broadcasted_iota(jnp.int32, (NH, H), 0)
    c = jax.lax.broadcasted_iota(jnp.int32, (NH, H), 1)
    return (r == c // HD).astype(jnp.float32)


def _exv_kernel(cell_ref, lab_ref, tis_ref, wqT_ref, bq_ref,
                wkT_ref, bk_ref, wvT_ref, bv_ref,
                exv_ref, dn_ref, dn_scr, q_scr):
    i = pl.program_id(0)

    @pl.when(i == 0)
    def _():
        dn_scr[...] = jnp.zeros_like(dn_scr)
        q_scr[...] = (
            jnp.dot(tis_ref[...], wqT_ref[...],
                    preferred_element_type=jnp.float32) + bq_ref[...]
        ).astype(jnp.bfloat16)

    K = (jnp.dot(cell_ref[...], wkT_ref[...],
                 preferred_element_type=jnp.float32) + bk_ref[...])
    lab = lab_ref[0]
    tid = jax.lax.broadcasted_iota(jnp.int32, (1, NT), 1)
    oh = (lab == tid).astype(jnp.bfloat16)
    qg = jnp.dot(oh, q_scr[...],
                 preferred_element_type=jnp.float32)
    s = jnp.dot(qg * K, _head_expand().T,
                preferred_element_type=jnp.float32) * SCALE
    ex = jnp.exp(s)
    V = (jnp.dot(cell_ref[...], wvT_ref[...],
                 preferred_element_type=jnp.float32) + bv_ref[...])
    exR = jnp.dot(ex, _head_expand(),
                  preferred_element_type=jnp.float32)
    exv_ref[...] = V * exR
    sel = (jax.lax.broadcasted_iota(jnp.int32, (NH, 16), 0)
           == jax.lax.broadcasted_iota(jnp.int32, (NH, 16), 1)
           ).astype(jnp.float32)
    col8 = (jax.lax.broadcasted_iota(jnp.int32, (BC, 16), 1) == 8
            ).astype(jnp.float32)
    dnr = jnp.dot(ex, sel, preferred_element_type=jnp.float32) + col8
    dn_scr[...] += jax.lax.dot_general(
        oh, dnr.astype(jnp.bfloat16), (((0,), (0,)), ((), ())),
        preferred_element_type=jnp.float32)

    @pl.when(i == NB - 1)
    def _():
        dn_ref[...] = dn_scr[...]


def _cell_kernel(cell_ref, lab_ref, att2_ref, dn_ref, tis_ref,
                 woT_ref, bo_ref, tdwvT_ref, tdbv_ref, tdwoT_ref, tdbo_ref,
                 tg_ref, tb_ref, g_ref, b_ref,
                 tout_ref, out_ref, td_scr):
    i = pl.program_id(0)

    @pl.when(i == 0)
    def _():
        dn = dn_ref[...]
        att_raw = att2_ref[0] + att2_ref[1]
        counts = dn[:, 8:9]
        mask = counts > 0.5
        denom = dn[:, :NH]
        denom = jnp.where(denom == 0.0, 1.0, denom)
        rep = jnp.dot(1.0 / denom, _head_expand(),
                      preferred_element_type=jnp.float32)
        att = att_raw * rep
        att_o = (jnp.dot(att, woT_ref[...],
                         preferred_element_type=jnp.float32) + bo_ref[...])
        tis = tis_ref[...]
        t_upd = jnp.where(mask, att_o, tis)
        td_v = (jnp.dot(t_upd, tdwvT_ref[...],
                        preferred_element_type=jnp.float32) + tdbv_ref[...])
        td_scr[...] = (jnp.dot(td_v, tdwoT_ref[...],
                               preferred_element_type=jnp.float32)
                       + tdbo_ref[...]).astype(jnp.bfloat16)
        tout_ref[...] = _ln(tis + t_upd, tg_ref[...], tb_ref[...])

    lab = lab_ref[0]
    tid = jax.lax.broadcasted_iota(jnp.int32, (1, NT), 1)
    oh = (lab == tid).astype(jnp.bfloat16)
    G = jnp.dot(oh, td_scr[...], preferred_element_type=jnp.float32)
    out_ref[...] = _ln(cell_ref[...] + G, g_ref[...], b_ref[...])


def _full(shape):
    return pl.BlockSpec(shape, lambda i: tuple(0 for _ in shape))


def kernel(cell_features, tissue_features, cluster_labels, tissue_batch,
           bu_Wq, bu_bq, bu_Wk, bu_bk, bu_Wv, bu_bv, bu_Wo, bu_bo,
           td_Wq, td_bq, td_Wk, td_bk, td_Wv, td_bv, td_Wo, td_bo,
           cell_ln_g, cell_ln_b, tissue_ln_g, tissue_ln_b):
    lab_col = cluster_labels.reshape(NB, BC, 1)

    exv, dn = pl.pallas_call(
        _exv_kernel,
        grid=(NB,),
        in_specs=[
            pl.BlockSpec((BC, H), lambda i: (i, 0)),
            pl.BlockSpec((1, BC, 1), lambda i: (i, 0, 0)),
            _full((NT, H)),
            _full((H, H)),
            _full((1, H)),
            _full((H, H)),
            _full((1, H)),
            _full((H, H)),
            _full((1, H)),
        ],
        out_specs=[
            pl.BlockSpec((BC, H), lambda i: (i, 0)),
            pl.BlockSpec((NT, 16), lambda i: (0, 0)),
        ],
        out_shape=[
            jax.ShapeDtypeStruct((NC, H), jnp.float32),
            jax.ShapeDtypeStruct((NT, 16), jnp.float32),
        ],
        scratch_shapes=[
            pltpu.VMEM((NT, 16), jnp.float32),
            pltpu.VMEM((NT, H), jnp.bfloat16),
        ],
    )(cell_features, lab_col, tissue_features,
      bu_Wq.T, bu_bq.reshape(1, H),
      bu_Wk.T, bu_bk.reshape(1, H),
      bu_Wv.T, bu_bv.reshape(1, H))

    att2 = _sc_scatter(exv, cluster_labels)

    tissue_out, cell_out = pl.pallas_call(
        _cell_kernel,
        grid=(NB,),
        in_specs=[
            pl.BlockSpec((BC, H), lambda i: (i, 0)),
            pl.BlockSpec((1, BC, 1), lambda i: (i, 0, 0)),
            _full((2, NT, H)),
            _full((NT, 16)),
            _full((NT, H)),
            _full((H, H)),
            _full((1, H)),
            _full((H, H)),
            _full((1, H)),
            _full((H, H)),
            _full((1, H)),
            _full((1, H)),
            _full((1, H)),
            _full((1, H)),
            _full((1, H)),
        ],
        out_specs=[
            pl.BlockSpec((NT, H), lambda i: (0, 0)),
            pl.BlockSpec((BC, H), lambda i: (i, 0)),
        ],
        out_shape=[
            jax.ShapeDtypeStruct((NT, H), jnp.float32),
            jax.ShapeDtypeStruct((NC, H), jnp.float32),
        ],
        scratch_shapes=[pltpu.VMEM((NT, H), jnp.bfloat16)],
    )(cell_features, lab_col, att2, dn, tissue_features,
      bu_Wo.T, bu_bo.reshape(1, H),
      td_Wv.T, td_bv.reshape(1, H), td_Wo.T, td_bo.reshape(1, H),
      tissue_ln_g.reshape(1, H), tissue_ln_b.reshape(1, H),
      cell_ln_g.reshape(1, H), cell_ln_b.reshape(1, H))

    return cell_out, tissue_out
